# baseline (speedup 1.0000x reference)
import functools

import jax
import jax.numpy as jnp
from jax import lax
from jax.experimental import pallas as pl
from jax.experimental.pallas import tpu as pltpu
from jax.experimental.pallas import tpu_sc as plsc

NU = 1000000
NI = 1000000
D = 32
B = 16384
R = 128

_NC, _NS = 2, 16
_NW = _NC * _NS
_BPW = B // _NW
_G = 8
_NGRP = _BPW // _G
_CMAX = (NU - 1) >> 7


def _sc_body(ug_hbm, ui_hbm, um_hbm, im_hbm, uidx_hbm, iidx_hbm, out,
             uvals, ivals, big0, big1, rows0, rows1, sem0, sem1, wsem0, wsem1):
    wid = lax.axis_index("s") * _NC + lax.axis_index("c")
    base = wid * _BPW

    pltpu.sync_copy(uidx_hbm.at[pl.ds(base, _BPW)], uvals.at[pl.ds(0, _BPW)])
    pltpu.sync_copy(iidx_hbm.at[pl.ds(base, _BPW)], ivals.at[pl.ds(0, _BPW)])

    ei = lax.iota(jnp.int32, 16)
    eic = jnp.minimum(ei, _G - 1)
    msk = ei < _G

    def fire(tab, vec, buf, sem):
        for e in range(_G):
            c = jnp.maximum(jnp.minimum(vec[e] >> 7, _CMAX), 0)
            off = pl.multiple_of(c * 128, 128)
            pltpu.async_copy(tab.at[:, pl.ds(off, 128)],
                             buf.at[pl.ds(e * D, D)], sem)

    def wait(buf, sem):
        for e in range(_G):
            pltpu.make_async_copy(ug_hbm.at[:, pl.ds(0, 128)],
                                  buf.at[pl.ds(e * D, D)], sem).wait()

    def extract(buf, lvec, t, rows):
        for d in range(D):
            dsp = jnp.full((16,), d, jnp.int32)
            vals = plsc.load_gather(buf, [eic * D + dsp, lvec])
            plsc.store_scatter(rows, [ei, dsp + t * D], vals, mask=msk)

    fire(ug_hbm, uvals[pl.ds(0, 16)], big0, sem0)

    def wait_rows(rows, wsem):
        pltpu.make_async_copy(rows.at[pl.ds(0, _G)],
                              out.at[pl.ds(0, _G)], wsem).wait()

    def body(g, rows, wsem, first, last):
        uvec = uvals[pl.ds(g * _G, 16)]
        ivec = ivals[pl.ds(g * _G, 16)]
        unext = uvals[pl.ds(g * _G + _G, 16)]
        lu = jnp.bitwise_and(uvec, 127)
        li = jnp.bitwise_and(ivec, 127)
        if not first:
            wait_rows(rows, wsem)
        fire(ui_hbm, ivec, big1, sem1)
        wait(big0, sem0)
        extract(big0, lu, 0, rows)
        fire(um_hbm, uvec, big0, sem0)
        wait(big1, sem1)
        extract(big1, li, 1, rows)
        fire(im_hbm, ivec, big1, sem1)
        wait(big0, sem0)
        extract(big0, lu, 2, rows)
        if not last:
            fire(ug_hbm, unext, big0, sem0)
        wait(big1, sem1)
        extract(big1, li, 3, rows)
        pltpu.async_copy(rows.at[pl.ds(0, _G)],
                         out.at[pl.ds(base + g * _G, _G)], wsem)

    def group2(h, carry):
        body(2 * h, rows0, wsem0, False, False)
        body(2 * h + 1, rows1, wsem1, False, False)
        return carry

    body(0, rows0, wsem0, True, False)
    body(1, rows1, wsem1, True, False)
    lax.fori_loop(1, _NGRP // 2 - 1, group2, 0)
    body(_NGRP - 2, rows0, wsem0, False, False)
    body(_NGRP - 1, rows1, wsem1, False, True)
    wait_rows(rows0, wsem0)
    wait_rows(rows1, wsem1)


@functools.lru_cache(maxsize=1)
def _sc_gather():
    mesh = plsc.VectorSubcoreMesh(core_axis_name="c", subcore_axis_name="s",
                                  num_cores=_NC, num_subcores=_NS)
    return pl.kernel(
        _sc_body,
        out_type=jax.ShapeDtypeStruct((B, R), jnp.float32),
        mesh=mesh,
        scratch_types=[
            pltpu.VMEM((_BPW + 16,), jnp.int32),
            pltpu.VMEM((_BPW + 16,), jnp.int32),
            pltpu.VMEM((_G * D, 128), jnp.float32),
            pltpu.VMEM((_G * D, 128), jnp.float32),
            pltpu.VMEM((16, R), jnp.float32),
            pltpu.VMEM((16, R), jnp.float32),
            pltpu.SemaphoreType.DMA,
            pltpu.SemaphoreType.DMA,
            pltpu.SemaphoreType.DMA,
            pltpu.SemaphoreType.DMA,
        ],
        compiler_params=pltpu.CompilerParams(needs_layout_passes=False),
    )


def _tc_body(x, w0t, b0, w1t, b1, w2t, b2, w3t, b3, wpg, wpm, bp, out):
    xv = x[...]
    g_u = xv[:, 0 * D:1 * D]
    g_i = xv[:, 1 * D:2 * D]
    m_u = xv[:, 2 * D:3 * D]
    m_i = xv[:, 3 * D:4 * D]
    gmf = g_u * g_i
    w0 = w0t[...]
    h = jnp.maximum(m_u @ w0[:D] + m_i @ w0[D:] + b0[...], 0.0)
    h = jnp.maximum(h @ w1t[...] + b1[...], 0.0)
    h = jnp.maximum(h @ w2t[...] + b2[...], 0.0)
    h = jnp.maximum(h @ w3t[...] + b3[...], 0.0)
    p = gmf @ wpg[...] + h @ wpm[...] + bp[...]
    out[...] = 1.0 / (1.0 + jnp.exp(-p))


def kernel(user_indices, item_indices, embed_user_gmf, embed_item_gmf,
           embed_user_mlp, embed_item_mlp, W0, b0, W1, b1, W2, b2, W3, b3,
           Wp, bp):
    uidx = user_indices.astype(jnp.int32)
    iidx = item_indices.astype(jnp.int32)

    packed = _sc_gather()(embed_user_gmf.T, embed_item_gmf.T,
                          embed_user_mlp.T, embed_item_mlp.T, uidx, iidx)

    blk = 2048
    grid = B // blk

    def full(shape):
        return pl.BlockSpec(shape, lambda i: tuple(0 for _ in shape))

    w0t = W0.T
    w1t = W1.T
    w2t = W2.T
    w3t = W3.T
    wpg = Wp[:, :D].T
    wpm = Wp[:, D:].T

    out = pl.pallas_call(
        _tc_body,
        grid=(grid,),
        in_specs=[
            pl.BlockSpec((blk, R), lambda i: (i, 0)),
            full((2 * D, 2 * D)), full((1, 2 * D)),
            full((2 * D, 32)), full((1, 32)),
            full((32, 16)), full((1, 16)),
            full((16, 8)), full((1, 8)),
            full((D, 1)), full((8, 1)), full((1, 1)),
        ],
        out_specs=pl.BlockSpec((blk, 1), lambda i: (i, 0)),
        out_shape=jax.ShapeDtypeStruct((B, 1), jnp.float32),
    )(packed,
      w0t, b0.reshape(1, -1), w1t, b1.reshape(1, -1), w2t, b2.reshape(1, -1),
      w3t, b3.reshape(1, -1), wpg, wpm, bp.reshape(1, 1))

    return out.reshape(B)


# speedup vs baseline: 1.0125x; 1.0125x over previous
import functools

import jax
import jax.numpy as jnp
from jax import lax
from jax.experimental import pallas as pl
from jax.experimental.pallas import tpu as pltpu
from jax.experimental.pallas import tpu_sc as plsc

NU = 1000000
NI = 1000000
D = 32
B = 16384
R = 128

_NC, _NS = 2, 16
_NW = _NC * _NS
_BPW = B // _NW
_G = 8
_NGRP = _BPW // _G
_CMAX = (NU - 1) >> 7


def _sc_body(ug_hbm, ui_hbm, um_hbm, im_hbm, uidx_hbm, iidx_hbm, out,
             uvals, ivals, big0, big1, rows, sem0, sem1):
    wid = lax.axis_index("s") * _NC + lax.axis_index("c")
    base = wid * _BPW

    pltpu.sync_copy(uidx_hbm.at[pl.ds(base, _BPW)], uvals.at[pl.ds(0, _BPW)])
    pltpu.sync_copy(iidx_hbm.at[pl.ds(base, _BPW)], ivals.at[pl.ds(0, _BPW)])

    ei = lax.iota(jnp.int32, 16)
    eic = jnp.minimum(ei, _G - 1)
    msk = ei < _G

    def fire(tab, vec, buf, sem):
        for e in range(_G):
            c = jnp.maximum(jnp.minimum(vec[e] >> 7, _CMAX), 0)
            off = pl.multiple_of(c * 128, 128)
            pltpu.async_copy(tab.at[:, pl.ds(off, 128)],
                             buf.at[pl.ds(e * D, D)], sem)

    def wait(buf, sem):
        for e in range(_G):
            pltpu.make_async_copy(ug_hbm.at[:, pl.ds(0, 128)],
                                  buf.at[pl.ds(e * D, D)], sem).wait()

    def extract(buf, lvec, t):
        for d in range(D):
            dsp = jnp.full((16,), d, jnp.int32)
            vals = plsc.load_gather(buf, [eic * D + dsp, lvec])
            plsc.store_scatter(rows, [ei, dsp + t * D], vals, mask=msk)

    fire(ug_hbm, uvals[pl.ds(0, 16)], big0, sem0)

    def group(g, carry):
        uvec = uvals[pl.ds(g * _G, 16)]
        ivec = ivals[pl.ds(g * _G, 16)]
        unext = uvals[pl.ds(g * _G + _G, 16)]
        lu = jnp.bitwise_and(uvec, 127)
        li = jnp.bitwise_and(ivec, 127)
        fire(ui_hbm, ivec, big1, sem1)
        wait(big0, sem0)
        extract(big0, lu, 0)
        fire(um_hbm, uvec, big0, sem0)
        wait(big1, sem1)
        extract(big1, li, 1)
        fire(im_hbm, ivec, big1, sem1)
        wait(big0, sem0)
        extract(big0, lu, 2)
        fire(ug_hbm, unext, big0, sem0)
        wait(big1, sem1)
        extract(big1, li, 3)
        pltpu.sync_copy(rows.at[pl.ds(0, _G)],
                        out.at[pl.ds(base + g * _G, _G)])
        return carry

    lax.fori_loop(0, _NGRP, group, 0)
    wait(big0, sem0)


@functools.lru_cache(maxsize=1)
def _sc_gather():
    mesh = plsc.VectorSubcoreMesh(core_axis_name="c", subcore_axis_name="s",
                                  num_cores=_NC, num_subcores=_NS)
    return pl.kernel(
        _sc_body,
        out_type=jax.ShapeDtypeStruct((B, R), jnp.float32),
        mesh=mesh,
        scratch_types=[
            pltpu.VMEM((_BPW + 16,), jnp.int32),
            pltpu.VMEM((_BPW + 16,), jnp.int32),
            pltpu.VMEM((_G * D, 128), jnp.float32),
            pltpu.VMEM((_G * D, 128), jnp.float32),
            pltpu.VMEM((16, R), jnp.float32),
            pltpu.SemaphoreType.DMA,
            pltpu.SemaphoreType.DMA,
        ],
        compiler_params=pltpu.CompilerParams(needs_layout_passes=False),
    )


def _tc_body(x, w0t, b0, w1t, b1, w2t, b2, w3t, b3, wpg, wpm, bp, out):
    xv = x[...]
    g_u = xv[:, 0 * D:1 * D]
    g_i = xv[:, 1 * D:2 * D]
    m_u = xv[:, 2 * D:3 * D]
    m_i = xv[:, 3 * D:4 * D]
    gmf = g_u * g_i
    w0 = w0t[...]
    h = jnp.maximum(m_u @ w0[:D] + m_i @ w0[D:] + b0[...], 0.0)
    h = jnp.maximum(h @ w1t[...] + b1[...], 0.0)
    h = jnp.maximum(h @ w2t[...] + b2[...], 0.0)
    h = jnp.maximum(h @ w3t[...] + b3[...], 0.0)
    p = gmf @ wpg[...] + h @ wpm[...] + bp[...]
    out[...] = 1.0 / (1.0 + jnp.exp(-p))


def kernel(user_indices, item_indices, embed_user_gmf, embed_item_gmf,
           embed_user_mlp, embed_item_mlp, W0, b0, W1, b1, W2, b2, W3, b3,
           Wp, bp):
    uidx = user_indices.astype(jnp.int32)
    iidx = item_indices.astype(jnp.int32)

    packed = _sc_gather()(embed_user_gmf.T, embed_item_gmf.T,
                          embed_user_mlp.T, embed_item_mlp.T, uidx, iidx)

    blk = 2048
    grid = B // blk

    def full(shape):
        return pl.BlockSpec(shape, lambda i: tuple(0 for _ in shape))

    w0t = W0.T
    w1t = W1.T
    w2t = W2.T
    w3t = W3.T
    wpg = Wp[:, :D].T
    wpm = Wp[:, D:].T

    out = pl.pallas_call(
        _tc_body,
        grid=(grid,),
        in_specs=[
            pl.BlockSpec((blk, R), lambda i: (i, 0)),
            full((2 * D, 2 * D)), full((1, 2 * D)),
            full((2 * D, 32)), full((1, 32)),
            full((32, 16)), full((1, 16)),
            full((16, 8)), full((1, 8)),
            full((D, 1)), full((8, 1)), full((1, 1)),
        ],
        out_specs=pl.BlockSpec((blk, 1), lambda i: (i, 0)),
        out_shape=jax.ShapeDtypeStruct((B, 1), jnp.float32),
    )(packed,
      w0t, b0.reshape(1, -1), w1t, b1.reshape(1, -1), w2t, b2.reshape(1, -1),
      w3t, b3.reshape(1, -1), wpg, wpm, bp.reshape(1, 1))

    return out.reshape(B)
